# baseline (speedup 1.0000x reference)
import functools

import jax
import jax.numpy as jnp
from jax import lax
from jax.experimental import pallas as pl
from jax.experimental.pallas import tpu as pltpu
from jax.experimental.pallas import tpu_sc as plsc

B = 4096
T = 50
TP = 64
D = 16
PR = TP * D // 128
NW = 32
CH = 128

SLOTS = B * TP
SPW = SLOTS // NW
NCH = SPW // CH
HALF = SPW // 2

BPW = B // NW
WPT = 26 * BPW // CH
WSPW = 26 * BPW

_mesh = plsc.VectorSubcoreMesh(core_axis_name="c", subcore_axis_name="s")
_sc_params = pltpu.CompilerParams(use_tc_tiling_on_sc=False)


def _wid():
    return lax.axis_index("s") * 2 + lax.axis_index("c")


@functools.partial(
    pl.kernel,
    out_type=(
        jax.ShapeDtypeStruct((SLOTS, D), jnp.float32),
        jax.ShapeDtypeStruct((SLOTS, D), jnp.float32),
        jax.ShapeDtypeStruct((B * 2, D), jnp.float32),
        jax.ShapeDtypeStruct((B, D), jnp.float32),
        jax.ShapeDtypeStruct((B * 26, D), jnp.float32),
        jax.ShapeDtypeStruct((B * 26, D), jnp.float32),
    ),
    mesh=_mesh,
    compiler_params=_sc_params,
    scratch_types=[
        pltpu.VMEM((NCH, CH), jnp.int32),
        pltpu.VMEM((NCH, CH), jnp.int32),
        pltpu.VMEM((NCH, CH), jnp.int32),
        pltpu.VMEM((HALF, D), jnp.float32),
        pltpu.VMEM((2 * BPW // CH, CH), jnp.int32),
        pltpu.VMEM((BPW // CH, CH), jnp.int32),
        pltpu.VMEM((3 * BPW, D), jnp.float32),
        pltpu.VMEM((WPT, CH), jnp.int32),
        pltpu.VMEM((WPT, CH), jnp.int32),
        pltpu.SemaphoreType.DMA,
        pltpu.SemaphoreType.DMA,
        pltpu.SemaphoreType.DMA,
    ],
)
def _sc_gather(din_hbm, ids1_hbm, ids3_hbm, pos_hbm,
               tid_hbm, oid_hbm, wid_hbm, did_hbm,
               s1_hbm, s3_hbm, tgtf_hbm, othf_hbm, widef_hbm, deepf_hbm,
               pos_v, ids1_v, ids3_v, rows_v,
               tid_v, oid_v, to_v, wid_v, did_v,
               sem, sem2, sem3):
    w = _wid()
    pltpu.sync_copy(pos_hbm.at[w], pos_v)
    pltpu.sync_copy(tid_hbm.at[w], tid_v)
    pltpu.sync_copy(oid_hbm.at[w], oid_v)
    pltpu.sync_copy(wid_hbm.at[w], wid_v)
    pltpu.sync_copy(did_hbm.at[w], did_v)

    for c in range(2):
        pltpu.async_copy(din_hbm.at[tid_v.at[c]],
                         to_v.at[pl.ds(c * CH, CH)], sem)
    pltpu.async_copy(din_hbm.at[oid_v.at[0]], to_v.at[pl.ds(2 * CH, CH)], sem)

    def fire_ids(c, carry):
        pltpu.async_copy(ids1_hbm.at[pos_v.at[c]], ids1_v.at[c], sem2)
        pltpu.async_copy(ids3_hbm.at[pos_v.at[c]], ids3_v.at[c], sem3)
        return carry
    lax.fori_loop(0, NCH, fire_ids, 0)
    pltpu.make_async_copy(pos_hbm.at[w], ids1_v, sem2).wait()

    for i1, out_hbm in ((0, s1_hbm), (1, s3_hbm)):
        ids_v = ids1_v if i1 == 0 else ids3_v
        for h in range(2):
            def fire_rows(c, carry, ids_v=ids_v, h=h):
                pltpu.async_copy(din_hbm.at[ids_v.at[c]],
                                 rows_v.at[pl.ds((c - h * (NCH // 2)) * CH, CH)],
                                 sem2)
                return carry
            lax.fori_loop(h * (NCH // 2), (h + 1) * (NCH // 2), fire_rows, 0)
            pltpu.make_async_copy(s1_hbm.at[pl.ds(0, HALF)], rows_v, sem2).wait()
            pltpu.sync_copy(rows_v, out_hbm.at[pl.ds(w * SPW + h * HALF, HALF)])
        if i1 == 0:
            pltpu.make_async_copy(pos_hbm.at[w], ids3_v, sem3).wait()

    for j in range(2):
        ids_v = wid_v if j == 0 else did_v
        out_hbm = widef_hbm if j == 0 else deepf_hbm

        def fire_wd(c, carry, ids_v=ids_v):
            pltpu.async_copy(din_hbm.at[ids_v.at[c]],
                             rows_v.at[pl.ds(c * CH, CH)], sem2)
            return carry
        lax.fori_loop(0, WPT, fire_wd, 0)
        pltpu.make_async_copy(out_hbm.at[pl.ds(0, WSPW)],
                              rows_v.at[pl.ds(0, WSPW)], sem2).wait()
        pltpu.sync_copy(rows_v.at[pl.ds(0, WSPW)],
                        out_hbm.at[pl.ds(w * WSPW, WSPW)])

    pltpu.make_async_copy(tgtf_hbm.at[pl.ds(0, 3 * CH)], to_v, sem).wait()
    pltpu.sync_copy(to_v.at[pl.ds(0, 2 * BPW)],
                    tgtf_hbm.at[pl.ds(w * 2 * BPW, 2 * BPW)])
    pltpu.sync_copy(to_v.at[pl.ds(2 * BPW, BPW)],
                    othf_hbm.at[pl.ds(w * BPW, BPW)])


BB = 256
GRID = B // BB
BB2 = 1024
GRID2 = B // BB2


def _dice_k(x, alpha):
    p = jax.nn.sigmoid(x)
    return p * x + (1.0 - p) * alpha * x


def _rep_rows(x, n):
    return jnp.broadcast_to(x[:, None, :], (x.shape[0], n, x.shape[1])
                            ).reshape(x.shape[0] * n, x.shape[1])


def _tc_att_body(s1p_ref, s3p_ref, q1t_ref, q2t_ref, oth_ref, len_ref,
                 k1_ref, k2_ref, k3_ref, k4_ref, kaq1_ref, kaq2_ref,
                 ab1t_ref, aa1t_ref, kw2_ref, ab2t_ref, aa2t_ref, kw3_ref,
                 rmat_ref, smat_ref,
                 mb0_ref, mb1_ref, mb2_ref, mb3_ref, mb4_ref,
                 mpb1_ref, mpa1_ref, mw2_ref, mpb2_ref, mpa2_ref, mw3_ref,
                 scal_ref, out_ref):
    f32 = jnp.float32
    dot = functools.partial(jnp.dot, preferred_element_type=f32)
    s1p = s1p_ref[...]
    s3p = s3p_ref[...]
    q1t = q1t_ref[...]
    q2t = q2t_ref[...]

    q1p = _rep_rows(q1t, PR)
    q2p = _rep_rows(q2t, PR)
    qs1p = q1p * s1p
    qs3p = q2p * s3p

    h = (dot(s1p, k1_ref[...]) + dot(s3p, k2_ref[...])
         + dot(qs1p, k3_ref[...]) + dot(qs3p, k4_ref[...]))
    cqt = dot(q1t, kaq1_ref[...]) + dot(q2t, kaq2_ref[...])
    h = h + _rep_rows(cqt, PR) + ab1t_ref[...]
    h = _dice_k(h, aa1t_ref[...])
    h2 = _dice_k(dot(h, kw2_ref[...]) + ab2t_ref[...], aa2t_ref[...])
    scores = dot(h2, kw3_ref[...]) + scal_ref[0, 0]

    row8 = lax.broadcasted_iota(jnp.int32, (BB * PR, 8), 0) % PR
    colj = lax.broadcasted_iota(jnp.int32, (BB * PR, 8), 1)
    lenp = _rep_rows(len_ref[...], PR)
    mask = (row8 * 8 + colj) < lenp
    scores = jnp.where(mask, scores, -1e9)

    mglob = jnp.max(scores)
    e = jnp.exp(scores - mglob)
    rs = jnp.sum(e, axis=1, keepdims=True)
    denom = jnp.sum(rs.reshape(BB, PR, 1), axis=1)
    wp = e / _rep_rows(denom, PR)

    wE = dot(wp, rmat_ref[...])
    pe1 = jnp.sum((wE * s1p).reshape(BB, PR, 128), axis=1)
    pe3 = jnp.sum((wE * s3p).reshape(BB, PR, 128), axis=1)
    pooled1 = dot(pe1, smat_ref[...])
    pooled3 = dot(pe3, smat_ref[...])

    q1 = q1t[:, :D]
    q2 = q2t[:, :D]
    z = (dot(oth_ref[...], mb0_ref[...]) + dot(pooled1, mb1_ref[...])
         + dot(pooled3, mb2_ref[...]) + dot(q1, mb3_ref[...])
         + dot(q2, mb4_ref[...]) + mpb1_ref[...])
    z = _dice_k(z, mpa1_ref[...])
    z = _dice_k(dot(z, mw2_ref[...]) + mpb2_ref[...], mpa2_ref[...])
    out_ref[...] = jnp.sum(z * mw3_ref[...], axis=-1)[:, None]


def _tc_fin_body(dino_ref, widef_ref, deepf_ref,
                 lrw_ref, dw1_ref, db1_ref, dw2_ref, db2_ref, dw3_ref,
                 scal_ref, out_ref):
    f32 = jnp.float32
    dot = functools.partial(jnp.dot, preferred_element_type=f32)
    lr_o = jnp.sum(widef_ref[...] * lrw_ref[...], axis=-1, keepdims=True)

    hd = jnp.maximum(dot(deepf_ref[...], dw1_ref[...]) + db1_ref[...], 0.0)
    hd = jnp.maximum(dot(hd, dw2_ref[...]) + db2_ref[...], 0.0)
    deep_o = jnp.sum(hd * dw3_ref[...], axis=-1, keepdims=True)

    bias = scal_ref[0, 1] + scal_ref[0, 2] + scal_ref[0, 3]
    out_ref[...] = jax.nn.sigmoid(dino_ref[...] + lr_o + deep_o + bias)


def _full(shape):
    n = len(shape)
    return pl.BlockSpec(shape, lambda i, n=n: (0,) * n)


def kernel(params, seq_ids_1, seq_ids_3, cu_seqlens, target_ids, other_ids,
           wide_ids, deep_ids):
    f32 = jnp.float32
    cu = cu_seqlens.astype(jnp.int32)
    lengths = (cu[1:] - cu[:-1]).reshape(B, 1)
    total = seq_ids_1.shape[0]
    pos = jnp.clip(cu[:-1, None] + jnp.arange(TP, dtype=jnp.int32),
                   0, total - 1)
    pos3d = pos.reshape(NW, NCH, CH)

    V = params['din_table'].shape[0]
    cat_table = jnp.concatenate(
        [params['din_table'], params['wide_table'], params['deep_table']], 0)
    s1f, s3f, tgtf, othf, widef, deepf = _sc_gather(
        cat_table, seq_ids_1, seq_ids_3, pos3d,
        target_ids.reshape(NW, 2 * BPW // CH, CH),
        other_ids.reshape(NW, BPW // CH, CH),
        (wide_ids + V).reshape(NW, WPT, CH),
        (deep_ids + 2 * V).reshape(NW, WPT, CH))

    s1p = s1f.reshape(B * PR, 128)
    s3p = s3f.reshape(B * PR, 128)
    tgt = tgtf.reshape(B, 2, D)
    q1t = jnp.tile(tgt[:, 0, :], (1, 128 // D))
    q2t = jnp.tile(tgt[:, 1, :], (1, 128 // D))
    oth = othf
    widef2 = widef.reshape(B, 26 * D)
    deepf2 = deepf.reshape(B, 26 * D)

    W1 = params['att_W1']
    W1q, W1s, W1d, W1m = W1[0:32], W1[32:64], W1[64:96], W1[96:128]
    As = W1s - W1d
    Aq = W1q + W1d
    eye8 = jnp.eye(128 // D, dtype=f32)
    krn = lambda wgt: jnp.kron(eye8, wgt)
    tl = lambda v: jnp.tile(v.reshape(1, -1), (1, 128 // D))

    mlpW1 = params['mlp_W1']
    mb = [mlpW1[i * D:(i + 1) * D] for i in range(5)]

    r1 = lambda v: v.reshape(1, -1)
    scal = jnp.stack([params['att_b3'][0], params['mlp_b3'][0],
                      params['lr_b'][0], params['deep_b3'][0]]).reshape(1, 4)

    att_ins = [
        krn(As[:D]), krn(As[D:]), krn(W1m[:D]), krn(W1m[D:]),
        krn(Aq[:D]), krn(Aq[D:]),
        tl(params['att_b1']), tl(params['att_a1']), krn(params['att_W2']),
        tl(params['att_b2']), tl(params['att_a2']), krn(params['att_W3']),
        jnp.kron(eye8, jnp.ones((1, D), f32)),
        jnp.tile(jnp.eye(D, dtype=f32), (128 // D, 1)),
        mb[0], mb[1], mb[2], mb[3], mb[4],
        r1(params['mlp_b1']), r1(params['mlp_a1']), params['mlp_W2'],
        r1(params['mlp_b2']), r1(params['mlp_a2']), r1(params['mlp_W3'][:, 0]),
        scal,
    ]

    att_specs = [
        pl.BlockSpec((BB * PR, 128), lambda i: (i, 0)),
        pl.BlockSpec((BB * PR, 128), lambda i: (i, 0)),
        pl.BlockSpec((BB, 128), lambda i: (i, 0)),
        pl.BlockSpec((BB, 128), lambda i: (i, 0)),
        pl.BlockSpec((BB, D), lambda i: (i, 0)),
        pl.BlockSpec((BB, 1), lambda i: (i, 0)),
    ] + [_full(w.shape) for w in att_ins]

    din_o = pl.pallas_call(
        _tc_att_body,
        grid=(GRID,),
        in_specs=att_specs,
        out_specs=pl.BlockSpec((BB, 1), lambda i: (i, 0)),
        out_shape=jax.ShapeDtypeStruct((B, 1), jnp.float32),
    )(s1p, s3p, q1t, q2t, oth, lengths, *att_ins)

    fin_ins = [
        r1(params['lr_w'][:, 0]),
        params['deep_W1'], r1(params['deep_b1']), params['deep_W2'],
        r1(params['deep_b2']), r1(params['deep_W3'][:, 0]),
        scal,
    ]
    fin_specs = [
        pl.BlockSpec((BB2, 1), lambda i: (i, 0)),
        pl.BlockSpec((BB2, 26 * D), lambda i: (i, 0)),
        pl.BlockSpec((BB2, 26 * D), lambda i: (i, 0)),
    ] + [_full(w.shape) for w in fin_ins]

    out = pl.pallas_call(
        _tc_fin_body,
        grid=(GRID2,),
        in_specs=fin_specs,
        out_specs=pl.BlockSpec((BB2, 1), lambda i: (i, 0)),
        out_shape=jax.ShapeDtypeStruct((B, 1), jnp.float32),
    )(din_o, widef2, deepf2, *fin_ins)
    return out


# speedup vs baseline: 1.1441x; 1.1441x over previous
import functools

import jax
import jax.numpy as jnp
from jax import lax
from jax.experimental import pallas as pl
from jax.experimental.pallas import tpu as pltpu
from jax.experimental.pallas import tpu_sc as plsc

B = 4096
T = 50
TP = 64
D = 16
PR = TP * D // 128
NW = 32
CH = 128

SLOTS = B * TP
SPW = SLOTS // NW
NCH = SPW // CH
HALF = SPW // 2

BPW = B // NW
WPT = 26 * BPW // CH
WSPW = 26 * BPW

_mesh = plsc.VectorSubcoreMesh(core_axis_name="c", subcore_axis_name="s")
_sc_params = pltpu.CompilerParams(use_tc_tiling_on_sc=False)


def _wid():
    return lax.axis_index("s") * 2 + lax.axis_index("c")


@functools.partial(
    pl.kernel,
    out_type=(
        jax.ShapeDtypeStruct((SLOTS, D), jnp.float32),
        jax.ShapeDtypeStruct((SLOTS, D), jnp.float32),
        jax.ShapeDtypeStruct((B * 2, D), jnp.float32),
        jax.ShapeDtypeStruct((B, D), jnp.float32),
        jax.ShapeDtypeStruct((B * 26, D), jnp.float32),
        jax.ShapeDtypeStruct((B * 26, D), jnp.float32),
    ),
    mesh=_mesh,
    compiler_params=_sc_params,
    scratch_types=[
        pltpu.VMEM((NCH, CH), jnp.int32),
        pltpu.VMEM((NCH, CH), jnp.int32),
        pltpu.VMEM((NCH, CH), jnp.int32),
        pltpu.VMEM((HALF, D), jnp.float32),
        pltpu.VMEM((2 * BPW // CH, CH), jnp.int32),
        pltpu.VMEM((BPW // CH, CH), jnp.int32),
        pltpu.VMEM((3 * BPW, D), jnp.float32),
        pltpu.VMEM((WPT, CH), jnp.int32),
        pltpu.VMEM((WPT, CH), jnp.int32),
        pltpu.SemaphoreType.DMA,
        pltpu.SemaphoreType.DMA,
        pltpu.SemaphoreType.DMA,
    ],
)
def _sc_gather(din_hbm, wide_t_hbm, deep_t_hbm, ids1_hbm, ids3_hbm, pos_hbm,
               tid_hbm, oid_hbm, wid_hbm, did_hbm,
               s1_hbm, s3_hbm, tgtf_hbm, othf_hbm, widef_hbm, deepf_hbm,
               pos_v, ids1_v, ids3_v, rows_v,
               tid_v, oid_v, to_v, wid_v, did_v,
               sem, sem2, sem3):
    w = _wid()
    pltpu.sync_copy(pos_hbm.at[w], pos_v)
    pltpu.sync_copy(tid_hbm.at[w], tid_v)
    pltpu.sync_copy(oid_hbm.at[w], oid_v)
    pltpu.sync_copy(wid_hbm.at[w], wid_v)
    pltpu.sync_copy(did_hbm.at[w], did_v)

    for c in range(2):
        pltpu.async_copy(din_hbm.at[tid_v.at[c]],
                         to_v.at[pl.ds(c * CH, CH)], sem)
    pltpu.async_copy(din_hbm.at[oid_v.at[0]], to_v.at[pl.ds(2 * CH, CH)], sem)

    def fire_ids(c, carry):
        pltpu.async_copy(ids1_hbm.at[pos_v.at[c]], ids1_v.at[c], sem2)
        pltpu.async_copy(ids3_hbm.at[pos_v.at[c]], ids3_v.at[c], sem3)
        return carry
    lax.fori_loop(0, NCH, fire_ids, 0)
    pltpu.make_async_copy(pos_hbm.at[w], ids1_v, sem2).wait()

    for i1, out_hbm in ((0, s1_hbm), (1, s3_hbm)):
        ids_v = ids1_v if i1 == 0 else ids3_v
        for h in range(2):
            def fire_rows(c, carry, ids_v=ids_v, h=h):
                pltpu.async_copy(din_hbm.at[ids_v.at[c]],
                                 rows_v.at[pl.ds((c - h * (NCH // 2)) * CH, CH)],
                                 sem2)
                return carry
            lax.fori_loop(h * (NCH // 2), (h + 1) * (NCH // 2), fire_rows, 0)
            pltpu.make_async_copy(s1_hbm.at[pl.ds(0, HALF)], rows_v, sem2).wait()
            pltpu.sync_copy(rows_v, out_hbm.at[pl.ds(w * SPW + h * HALF, HALF)])
        if i1 == 0:
            pltpu.make_async_copy(pos_hbm.at[w], ids3_v, sem3).wait()

    for j in range(2):
        ids_v = wid_v if j == 0 else did_v
        table = wide_t_hbm if j == 0 else deep_t_hbm
        out_hbm = widef_hbm if j == 0 else deepf_hbm

        def fire_wd(c, carry, ids_v=ids_v, table=table):
            pltpu.async_copy(table.at[ids_v.at[c]],
                             rows_v.at[pl.ds(c * CH, CH)], sem2)
            return carry
        lax.fori_loop(0, WPT, fire_wd, 0)
        pltpu.make_async_copy(out_hbm.at[pl.ds(0, WSPW)],
                              rows_v.at[pl.ds(0, WSPW)], sem2).wait()
        pltpu.sync_copy(rows_v.at[pl.ds(0, WSPW)],
                        out_hbm.at[pl.ds(w * WSPW, WSPW)])

    pltpu.make_async_copy(tgtf_hbm.at[pl.ds(0, 3 * CH)], to_v, sem).wait()
    pltpu.sync_copy(to_v.at[pl.ds(0, 2 * BPW)],
                    tgtf_hbm.at[pl.ds(w * 2 * BPW, 2 * BPW)])
    pltpu.sync_copy(to_v.at[pl.ds(2 * BPW, BPW)],
                    othf_hbm.at[pl.ds(w * BPW, BPW)])


BB = 256
GRID = B // BB
BB2 = 1024
GRID2 = B // BB2


def _dice_k(x, alpha):
    p = jax.nn.sigmoid(x)
    return p * x + (1.0 - p) * alpha * x


def _rep_rows(x, n):
    return jnp.broadcast_to(x[:, None, :], (x.shape[0], n, x.shape[1])
                            ).reshape(x.shape[0] * n, x.shape[1])


def _tc_att_body(s1p_ref, s3p_ref, q1t_ref, q2t_ref, oth_ref, len_ref,
                 k1_ref, k2_ref, k3_ref, k4_ref, kaq1_ref, kaq2_ref,
                 ab1t_ref, aa1t_ref, kw2_ref, ab2t_ref, aa2t_ref, kw3_ref,
                 rmat_ref, smat_ref,
                 mb0_ref, mb1_ref, mb2_ref, mb3_ref, mb4_ref,
                 mpb1_ref, mpa1_ref, mw2_ref, mpb2_ref, mpa2_ref, mw3_ref,
                 scal_ref, out_ref):
    f32 = jnp.float32
    dot = functools.partial(jnp.dot, preferred_element_type=f32)
    s1p = s1p_ref[...]
    s3p = s3p_ref[...]
    q1t = q1t_ref[...]
    q2t = q2t_ref[...]

    q1p = _rep_rows(q1t, PR)
    q2p = _rep_rows(q2t, PR)
    qs1p = q1p * s1p
    qs3p = q2p * s3p

    h = (dot(s1p, k1_ref[...]) + dot(s3p, k2_ref[...])
         + dot(qs1p, k3_ref[...]) + dot(qs3p, k4_ref[...]))
    cqt = dot(q1t, kaq1_ref[...]) + dot(q2t, kaq2_ref[...])
    h = h + _rep_rows(cqt, PR) + ab1t_ref[...]
    h = _dice_k(h, aa1t_ref[...])
    h2 = _dice_k(dot(h, kw2_ref[...]) + ab2t_ref[...], aa2t_ref[...])
    scores = dot(h2, kw3_ref[...]) + scal_ref[0, 0]

    row8 = lax.broadcasted_iota(jnp.int32, (BB * PR, 8), 0) % PR
    colj = lax.broadcasted_iota(jnp.int32, (BB * PR, 8), 1)
    lenp = _rep_rows(len_ref[...], PR)
    mask = (row8 * 8 + colj) < lenp
    scores = jnp.where(mask, scores, -1e9)

    mglob = jnp.max(scores)
    e = jnp.exp(scores - mglob)
    rs = jnp.sum(e, axis=1, keepdims=True)
    denom = jnp.sum(rs.reshape(BB, PR, 1), axis=1)
    wp = e / _rep_rows(denom, PR)

    wE = dot(wp, rmat_ref[...])
    pe1 = jnp.sum((wE * s1p).reshape(BB, PR, 128), axis=1)
    pe3 = jnp.sum((wE * s3p).reshape(BB, PR, 128), axis=1)
    pooled1 = dot(pe1, smat_ref[...])
    pooled3 = dot(pe3, smat_ref[...])

    q1 = q1t[:, :D]
    q2 = q2t[:, :D]
    z = (dot(oth_ref[...], mb0_ref[...]) + dot(pooled1, mb1_ref[...])
         + dot(pooled3, mb2_ref[...]) + dot(q1, mb3_ref[...])
         + dot(q2, mb4_ref[...]) + mpb1_ref[...])
    z = _dice_k(z, mpa1_ref[...])
    z = _dice_k(dot(z, mw2_ref[...]) + mpb2_ref[...], mpa2_ref[...])
    out_ref[...] = jnp.sum(z * mw3_ref[...], axis=-1)[:, None]


def _tc_fin_body(dino_ref, widef_ref, deepf_ref,
                 lrw_ref, dw1_ref, db1_ref, dw2_ref, db2_ref, dw3_ref,
                 scal_ref, out_ref):
    f32 = jnp.float32
    dot = functools.partial(jnp.dot, preferred_element_type=f32)
    lr_o = jnp.sum(widef_ref[...] * lrw_ref[...], axis=-1, keepdims=True)

    hd = jnp.maximum(dot(deepf_ref[...], dw1_ref[...]) + db1_ref[...], 0.0)
    hd = jnp.maximum(dot(hd, dw2_ref[...]) + db2_ref[...], 0.0)
    deep_o = jnp.sum(hd * dw3_ref[...], axis=-1, keepdims=True)

    bias = scal_ref[0, 1] + scal_ref[0, 2] + scal_ref[0, 3]
    out_ref[...] = jax.nn.sigmoid(dino_ref[...] + lr_o + deep_o + bias)


def _full(shape):
    n = len(shape)
    return pl.BlockSpec(shape, lambda i, n=n: (0,) * n)


def kernel(params, seq_ids_1, seq_ids_3, cu_seqlens, target_ids, other_ids,
           wide_ids, deep_ids):
    f32 = jnp.float32
    cu = cu_seqlens.astype(jnp.int32)
    lengths = (cu[1:] - cu[:-1]).reshape(B, 1)
    total = seq_ids_1.shape[0]
    pos = jnp.clip(cu[:-1, None] + jnp.arange(TP, dtype=jnp.int32),
                   0, total - 1)
    pos3d = pos.reshape(NW, NCH, CH)

    s1f, s3f, tgtf, othf, widef, deepf = _sc_gather(
        params['din_table'], params['wide_table'], params['deep_table'],
        seq_ids_1, seq_ids_3, pos3d,
        target_ids.reshape(NW, 2 * BPW // CH, CH),
        other_ids.reshape(NW, BPW // CH, CH),
        wide_ids.reshape(NW, WPT, CH),
        deep_ids.reshape(NW, WPT, CH))

    s1p = s1f.reshape(B * PR, 128)
    s3p = s3f.reshape(B * PR, 128)
    tgt = tgtf.reshape(B, 2, D)
    q1t = jnp.tile(tgt[:, 0, :], (1, 128 // D))
    q2t = jnp.tile(tgt[:, 1, :], (1, 128 // D))
    oth = othf
    widef2 = widef.reshape(B, 26 * D)
    deepf2 = deepf.reshape(B, 26 * D)

    W1 = params['att_W1']
    W1q, W1s, W1d, W1m = W1[0:32], W1[32:64], W1[64:96], W1[96:128]
    As = W1s - W1d
    Aq = W1q + W1d
    eye8 = jnp.eye(128 // D, dtype=f32)
    krn = lambda wgt: jnp.kron(eye8, wgt)
    tl = lambda v: jnp.tile(v.reshape(1, -1), (1, 128 // D))

    mlpW1 = params['mlp_W1']
    mb = [mlpW1[i * D:(i + 1) * D] for i in range(5)]

    r1 = lambda v: v.reshape(1, -1)
    scal = jnp.stack([params['att_b3'][0], params['mlp_b3'][0],
                      params['lr_b'][0], params['deep_b3'][0]]).reshape(1, 4)

    att_ins = [
        krn(As[:D]), krn(As[D:]), krn(W1m[:D]), krn(W1m[D:]),
        krn(Aq[:D]), krn(Aq[D:]),
        tl(params['att_b1']), tl(params['att_a1']), krn(params['att_W2']),
        tl(params['att_b2']), tl(params['att_a2']), krn(params['att_W3']),
        jnp.kron(eye8, jnp.ones((1, D), f32)),
        jnp.tile(jnp.eye(D, dtype=f32), (128 // D, 1)),
        mb[0], mb[1], mb[2], mb[3], mb[4],
        r1(params['mlp_b1']), r1(params['mlp_a1']), params['mlp_W2'],
        r1(params['mlp_b2']), r1(params['mlp_a2']), r1(params['mlp_W3'][:, 0]),
        scal,
    ]

    att_specs = [
        pl.BlockSpec((BB * PR, 128), lambda i: (i, 0)),
        pl.BlockSpec((BB * PR, 128), lambda i: (i, 0)),
        pl.BlockSpec((BB, 128), lambda i: (i, 0)),
        pl.BlockSpec((BB, 128), lambda i: (i, 0)),
        pl.BlockSpec((BB, D), lambda i: (i, 0)),
        pl.BlockSpec((BB, 1), lambda i: (i, 0)),
    ] + [_full(w.shape) for w in att_ins]

    din_o = pl.pallas_call(
        _tc_att_body,
        grid=(GRID,),
        in_specs=att_specs,
        out_specs=pl.BlockSpec((BB, 1), lambda i: (i, 0)),
        out_shape=jax.ShapeDtypeStruct((B, 1), jnp.float32),
    )(s1p, s3p, q1t, q2t, oth, lengths, *att_ins)

    fin_ins = [
        r1(params['lr_w'][:, 0]),
        params['deep_W1'], r1(params['deep_b1']), params['deep_W2'],
        r1(params['deep_b2']), r1(params['deep_W3'][:, 0]),
        scal,
    ]
    fin_specs = [
        pl.BlockSpec((BB2, 1), lambda i: (i, 0)),
        pl.BlockSpec((BB2, 26 * D), lambda i: (i, 0)),
        pl.BlockSpec((BB2, 26 * D), lambda i: (i, 0)),
    ] + [_full(w.shape) for w in fin_ins]

    out = pl.pallas_call(
        _tc_fin_body,
        grid=(GRID2,),
        in_specs=fin_specs,
        out_specs=pl.BlockSpec((BB2, 1), lambda i: (i, 0)),
        out_shape=jax.ShapeDtypeStruct((B, 1), jnp.float32),
    )(din_o, widef2, deepf2, *fin_ins)
    return out


# speedup vs baseline: 1.2355x; 1.0798x over previous
import functools

import jax
import jax.numpy as jnp
from jax import lax
from jax.experimental import pallas as pl
from jax.experimental.pallas import tpu as pltpu
from jax.experimental.pallas import tpu_sc as plsc

B = 4096
T = 50
TP = 64
D = 16
PR = TP * D // 128
NW = 32
CH = 128

SLOTS = B * TP
SPW = SLOTS // NW
NCH = SPW // CH
HALF = SPW // 2

BPW = B // NW
WPT = 26 * BPW // CH
WSPW = 26 * BPW

_mesh = plsc.VectorSubcoreMesh(core_axis_name="c", subcore_axis_name="s")
_sc_params = pltpu.CompilerParams(use_tc_tiling_on_sc=False)


def _wid():
    return lax.axis_index("s") * 2 + lax.axis_index("c")


B_H = B // 2
SLOTS_H = B_H * TP
SPW_H = SLOTS_H // NW
NCH_H = SPW_H // CH
HALF_H = SPW_H // 2


def _make_seq_gather(with_small):
    outs = [
        jax.ShapeDtypeStruct((SLOTS_H, D), jnp.float32),
        jax.ShapeDtypeStruct((SLOTS_H, D), jnp.float32),
    ]
    scratch = [
        pltpu.VMEM((NCH_H, CH), jnp.int32),
        pltpu.VMEM((NCH_H, CH), jnp.int32),
        pltpu.VMEM((NCH_H, CH), jnp.int32),
        pltpu.VMEM((HALF_H, D), jnp.float32),
        pltpu.SemaphoreType.DMA,
    ]
    if with_small:
        outs += [jax.ShapeDtypeStruct((B * 2, D), jnp.float32),
                 jax.ShapeDtypeStruct((B, D), jnp.float32)]
        scratch += [pltpu.VMEM((2 * BPW // CH, CH), jnp.int32),
                    pltpu.VMEM((BPW // CH, CH), jnp.int32),
                    pltpu.VMEM((3 * BPW, D), jnp.float32),
                    pltpu.SemaphoreType.DMA]

    @functools.partial(pl.kernel, out_type=tuple(outs), mesh=_mesh,
                       compiler_params=_sc_params, scratch_types=scratch)
    def k(*refs):
        if with_small:
            (din_hbm, ids1_hbm, ids3_hbm, pos_hbm, tid_hbm, oid_hbm,
             s1_hbm, s3_hbm, tgtf_hbm, othf_hbm,
             pos_v, ids1_v, ids3_v, rows_v, sem2,
             tid_v, oid_v, to_v, sem) = refs
        else:
            (din_hbm, ids1_hbm, ids3_hbm, pos_hbm,
             s1_hbm, s3_hbm,
             pos_v, ids1_v, ids3_v, rows_v, sem2) = refs
        w = _wid()
        pltpu.sync_copy(pos_hbm.at[w], pos_v)
        if with_small:
            pltpu.sync_copy(tid_hbm.at[w], tid_v)
            pltpu.sync_copy(oid_hbm.at[w], oid_v)
            for c in range(2):
                pltpu.async_copy(din_hbm.at[tid_v.at[c]],
                                 to_v.at[pl.ds(c * CH, CH)], sem)
            pltpu.async_copy(din_hbm.at[oid_v.at[0]],
                             to_v.at[pl.ds(2 * CH, CH)], sem)

        def fire_ids(c, carry):
            pltpu.async_copy(ids1_hbm.at[pos_v.at[c]], ids1_v.at[c], sem2)
            pltpu.async_copy(ids3_hbm.at[pos_v.at[c]], ids3_v.at[c], sem2)
            return carry
        lax.fori_loop(0, NCH_H, fire_ids, 0)
        pltpu.make_async_copy(pos_hbm.at[w], ids1_v, sem2).wait()
        pltpu.make_async_copy(pos_hbm.at[w], ids3_v, sem2).wait()

        for i1, out_hbm in ((0, s1_hbm), (1, s3_hbm)):
            ids_v = ids1_v if i1 == 0 else ids3_v
            for h in range(2):
                def fire_rows(c, carry, ids_v=ids_v, h=h):
                    pltpu.async_copy(
                        din_hbm.at[ids_v.at[c]],
                        rows_v.at[pl.ds((c - h * (NCH_H // 2)) * CH, CH)],
                        sem2)
                    return carry
                lax.fori_loop(h * (NCH_H // 2), (h + 1) * (NCH_H // 2),
                              fire_rows, 0)
                pltpu.make_async_copy(s1_hbm.at[pl.ds(0, HALF_H)],
                                      rows_v, sem2).wait()
                pltpu.sync_copy(rows_v,
                                out_hbm.at[pl.ds(w * SPW_H + h * HALF_H,
                                                 HALF_H)])

        if with_small:
            pltpu.make_async_copy(tgtf_hbm.at[pl.ds(0, 3 * CH)],
                                  to_v, sem).wait()
            pltpu.sync_copy(to_v.at[pl.ds(0, 2 * BPW)],
                            tgtf_hbm.at[pl.ds(w * 2 * BPW, 2 * BPW)])
            pltpu.sync_copy(to_v.at[pl.ds(2 * BPW, BPW)],
                            othf_hbm.at[pl.ds(w * BPW, BPW)])
    return k


_sc_seq_gather_a = _make_seq_gather(True)
_sc_seq_gather_b = _make_seq_gather(False)


@functools.partial(
    pl.kernel,
    out_type=(
        jax.ShapeDtypeStruct((B * 26, D), jnp.float32),
        jax.ShapeDtypeStruct((B * 26, D), jnp.float32),
    ),
    mesh=_mesh,
    compiler_params=_sc_params,
    scratch_types=[
        pltpu.VMEM((WPT, CH), jnp.int32),
        pltpu.VMEM((WPT, CH), jnp.int32),
        pltpu.VMEM((WSPW, D), jnp.float32),
        pltpu.SemaphoreType.DMA,
    ],
)
def _sc_table_gather(wide_t_hbm, deep_t_hbm, wid_hbm, did_hbm,
                     widef_hbm, deepf_hbm,
                     wid_v, did_v, rows_v, sem):
    w = _wid()
    pltpu.sync_copy(wid_hbm.at[w], wid_v)
    pltpu.sync_copy(did_hbm.at[w], did_v)

    for j in range(2):
        ids_v = wid_v if j == 0 else did_v
        table = wide_t_hbm if j == 0 else deep_t_hbm
        out_hbm = widef_hbm if j == 0 else deepf_hbm

        def fire(c, carry, ids_v=ids_v, table=table):
            pltpu.async_copy(table.at[ids_v.at[c]],
                             rows_v.at[pl.ds(c * CH, CH)], sem)
            return carry
        lax.fori_loop(0, WPT, fire, 0)
        pltpu.make_async_copy(out_hbm.at[pl.ds(0, WSPW)], rows_v, sem).wait()
        pltpu.sync_copy(rows_v, out_hbm.at[pl.ds(w * WSPW, WSPW)])


BB = 256
GRID = B // BB
BB2 = 1024
GRID2 = B // BB2


def _dice_k(x, alpha):
    p = jax.nn.sigmoid(x)
    return p * x + (1.0 - p) * alpha * x


def _rep_rows(x, n):
    return jnp.broadcast_to(x[:, None, :], (x.shape[0], n, x.shape[1])
                            ).reshape(x.shape[0] * n, x.shape[1])


def _tc_att_body(s1p_ref, s3p_ref, q1t_ref, q2t_ref, oth_ref, len_ref,
                 k1_ref, k2_ref, k3_ref, k4_ref, kaq1_ref, kaq2_ref,
                 ab1t_ref, aa1t_ref, kw2_ref, ab2t_ref, aa2t_ref, kw3_ref,
                 rmat_ref, smat_ref,
                 mb0_ref, mb1_ref, mb2_ref, mb3_ref, mb4_ref,
                 mpb1_ref, mpa1_ref, mw2_ref, mpb2_ref, mpa2_ref, mw3_ref,
                 scal_ref, out_ref):
    f32 = jnp.float32
    dot = functools.partial(jnp.dot, preferred_element_type=f32)
    s1p = s1p_ref[...]
    s3p = s3p_ref[...]
    q1t = q1t_ref[...]
    q2t = q2t_ref[...]

    q1p = _rep_rows(q1t, PR)
    q2p = _rep_rows(q2t, PR)
    qs1p = q1p * s1p
    qs3p = q2p * s3p

    h = (dot(s1p, k1_ref[...]) + dot(s3p, k2_ref[...])
         + dot(qs1p, k3_ref[...]) + dot(qs3p, k4_ref[...]))
    cqt = dot(q1t, kaq1_ref[...]) + dot(q2t, kaq2_ref[...])
    h = h + _rep_rows(cqt, PR) + ab1t_ref[...]
    h = _dice_k(h, aa1t_ref[...])
    h2 = _dice_k(dot(h, kw2_ref[...]) + ab2t_ref[...], aa2t_ref[...])
    scores = dot(h2, kw3_ref[...]) + scal_ref[0, 0]

    row8 = lax.broadcasted_iota(jnp.int32, (BB * PR, 8), 0) % PR
    colj = lax.broadcasted_iota(jnp.int32, (BB * PR, 8), 1)
    lenp = _rep_rows(len_ref[...], PR)
    mask = (row8 * 8 + colj) < lenp
    scores = jnp.where(mask, scores, -1e9)

    mglob = jnp.max(scores)
    e = jnp.exp(scores - mglob)
    rs = jnp.sum(e, axis=1, keepdims=True)
    denom = jnp.sum(rs.reshape(BB, PR, 1), axis=1)
    wp = e / _rep_rows(denom, PR)

    wE = dot(wp, rmat_ref[...])
    pe1 = jnp.sum((wE * s1p).reshape(BB, PR, 128), axis=1)
    pe3 = jnp.sum((wE * s3p).reshape(BB, PR, 128), axis=1)
    pooled1 = dot(pe1, smat_ref[...])
    pooled3 = dot(pe3, smat_ref[...])

    q1 = q1t[:, :D]
    q2 = q2t[:, :D]
    z = (dot(oth_ref[...], mb0_ref[...]) + dot(pooled1, mb1_ref[...])
         + dot(pooled3, mb2_ref[...]) + dot(q1, mb3_ref[...])
         + dot(q2, mb4_ref[...]) + mpb1_ref[...])
    z = _dice_k(z, mpa1_ref[...])
    z = _dice_k(dot(z, mw2_ref[...]) + mpb2_ref[...], mpa2_ref[...])
    out_ref[...] = jnp.sum(z * mw3_ref[...], axis=-1)[:, None]


def _tc_fin_body(dino_ref, widef_ref, deepf_ref,
                 lrw_ref, dw1_ref, db1_ref, dw2_ref, db2_ref, dw3_ref,
                 scal_ref, out_ref):
    f32 = jnp.float32
    dot = functools.partial(jnp.dot, preferred_element_type=f32)
    lr_o = jnp.sum(widef_ref[...] * lrw_ref[...], axis=-1, keepdims=True)

    hd = jnp.maximum(dot(deepf_ref[...], dw1_ref[...]) + db1_ref[...], 0.0)
    hd = jnp.maximum(dot(hd, dw2_ref[...]) + db2_ref[...], 0.0)
    deep_o = jnp.sum(hd * dw3_ref[...], axis=-1, keepdims=True)

    bias = scal_ref[0, 1] + scal_ref[0, 2] + scal_ref[0, 3]
    out_ref[...] = jax.nn.sigmoid(dino_ref[...] + lr_o + deep_o + bias)


def _full(shape):
    n = len(shape)
    return pl.BlockSpec(shape, lambda i, n=n: (0,) * n)


def kernel(params, seq_ids_1, seq_ids_3, cu_seqlens, target_ids, other_ids,
           wide_ids, deep_ids):
    f32 = jnp.float32
    cu = cu_seqlens.astype(jnp.int32)
    lengths = (cu[1:] - cu[:-1]).reshape(B, 1)
    total = seq_ids_1.shape[0]
    pos = jnp.clip(cu[:-1, None] + jnp.arange(TP, dtype=jnp.int32),
                   0, total - 1)
    pos3d = pos.reshape(NW, NCH, CH)

    pos_h = pos.reshape(2, NW, NCH_H, CH)
    s1f0, s3f0, tgtf, othf = _sc_seq_gather_a(
        params['din_table'], seq_ids_1, seq_ids_3, pos_h[0],
        target_ids.reshape(NW, 2 * BPW // CH, CH),
        other_ids.reshape(NW, BPW // CH, CH))
    s1f1, s3f1 = _sc_seq_gather_b(
        params['din_table'], seq_ids_1, seq_ids_3, pos_h[1])

    widef, deepf = _sc_table_gather(
        params['wide_table'], params['deep_table'],
        wide_ids.reshape(NW, WPT, CH), deep_ids.reshape(NW, WPT, CH))

    tgt = tgtf.reshape(B, 2, D)
    q1t = jnp.tile(tgt[:, 0, :], (1, 128 // D))
    q2t = jnp.tile(tgt[:, 1, :], (1, 128 // D))
    oth = othf
    widef2 = widef.reshape(B, 26 * D)
    deepf2 = deepf.reshape(B, 26 * D)

    W1 = params['att_W1']
    W1q, W1s, W1d, W1m = W1[0:32], W1[32:64], W1[64:96], W1[96:128]
    As = W1s - W1d
    Aq = W1q + W1d
    eye8 = jnp.eye(128 // D, dtype=f32)
    krn = lambda wgt: jnp.kron(eye8, wgt)
    tl = lambda v: jnp.tile(v.reshape(1, -1), (1, 128 // D))

    mlpW1 = params['mlp_W1']
    mb = [mlpW1[i * D:(i + 1) * D] for i in range(5)]

    r1 = lambda v: v.reshape(1, -1)
    scal = jnp.stack([params['att_b3'][0], params['mlp_b3'][0],
                      params['lr_b'][0], params['deep_b3'][0]]).reshape(1, 4)

    att_ins = [
        krn(As[:D]), krn(As[D:]), krn(W1m[:D]), krn(W1m[D:]),
        krn(Aq[:D]), krn(Aq[D:]),
        tl(params['att_b1']), tl(params['att_a1']), krn(params['att_W2']),
        tl(params['att_b2']), tl(params['att_a2']), krn(params['att_W3']),
        jnp.kron(eye8, jnp.ones((1, D), f32)),
        jnp.tile(jnp.eye(D, dtype=f32), (128 // D, 1)),
        mb[0], mb[1], mb[2], mb[3], mb[4],
        r1(params['mlp_b1']), r1(params['mlp_a1']), params['mlp_W2'],
        r1(params['mlp_b2']), r1(params['mlp_a2']), r1(params['mlp_W3'][:, 0]),
        scal,
    ]

    att_specs = [
        pl.BlockSpec((BB * PR, 128), lambda i: (i, 0)),
        pl.BlockSpec((BB * PR, 128), lambda i: (i, 0)),
        pl.BlockSpec((BB, 128), lambda i: (i, 0)),
        pl.BlockSpec((BB, 128), lambda i: (i, 0)),
        pl.BlockSpec((BB, D), lambda i: (i, 0)),
        pl.BlockSpec((BB, 1), lambda i: (i, 0)),
    ] + [_full(w.shape) for w in att_ins]

    din_halves = []
    for hh, (s1f_h, s3f_h) in enumerate(((s1f0, s3f0), (s1f1, s3f1))):
        sl = slice(hh * B_H, (hh + 1) * B_H)
        din_halves.append(pl.pallas_call(
            _tc_att_body,
            grid=(B_H // BB,),
            in_specs=att_specs,
            out_specs=pl.BlockSpec((BB, 1), lambda i: (i, 0)),
            out_shape=jax.ShapeDtypeStruct((B_H, 1), jnp.float32),
        )(s1f_h.reshape(B_H * PR, 128), s3f_h.reshape(B_H * PR, 128),
          q1t[sl], q2t[sl], oth[sl], lengths[sl], *att_ins))
    din_o = jnp.concatenate(din_halves, axis=0)

    fin_ins = [
        r1(params['lr_w'][:, 0]),
        params['deep_W1'], r1(params['deep_b1']), params['deep_W2'],
        r1(params['deep_b2']), r1(params['deep_W3'][:, 0]),
        scal,
    ]
    fin_specs = [
        pl.BlockSpec((BB2, 1), lambda i: (i, 0)),
        pl.BlockSpec((BB2, 26 * D), lambda i: (i, 0)),
        pl.BlockSpec((BB2, 26 * D), lambda i: (i, 0)),
    ] + [_full(w.shape) for w in fin_ins]

    out = pl.pallas_call(
        _tc_fin_body,
        grid=(GRID2,),
        in_specs=fin_specs,
        out_specs=pl.BlockSpec((BB2, 1), lambda i: (i, 0)),
        out_shape=jax.ShapeDtypeStruct((B, 1), jnp.float32),
    )(din_o, widef2, deepf2, *fin_ins)
    return out


# speedup vs baseline: 1.2743x; 1.0314x over previous
import functools

import jax
import jax.numpy as jnp
from jax import lax
from jax.experimental import pallas as pl
from jax.experimental.pallas import tpu as pltpu
from jax.experimental.pallas import tpu_sc as plsc

B = 4096
T = 50
TP = 64
D = 16
PR = TP * D // 128
NW = 32
CH = 128

SLOTS = B * TP
SPW = SLOTS // NW
NCH = SPW // CH
HALF = SPW // 2

BPW = B // NW
WPT = 26 * BPW // CH
WSPW = 26 * BPW

_mesh = plsc.VectorSubcoreMesh(core_axis_name="c", subcore_axis_name="s")
_sc_params = pltpu.CompilerParams(use_tc_tiling_on_sc=False)


def _wid():
    return lax.axis_index("s") * 2 + lax.axis_index("c")


@functools.partial(
    pl.kernel,
    out_type=(
        jax.ShapeDtypeStruct((SLOTS, D), jnp.float32),
        jax.ShapeDtypeStruct((SLOTS, D), jnp.float32),
        jax.ShapeDtypeStruct((B * 2, D), jnp.float32),
        jax.ShapeDtypeStruct((B, D), jnp.float32),
    ),
    mesh=_mesh,
    compiler_params=_sc_params,
    scratch_types=[
        pltpu.VMEM((NCH, CH), jnp.int32),
        pltpu.VMEM((NCH, CH), jnp.int32),
        pltpu.VMEM((NCH, CH), jnp.int32),
        pltpu.VMEM((HALF, D), jnp.float32),
        pltpu.VMEM((2 * BPW // CH, CH), jnp.int32),
        pltpu.VMEM((BPW // CH, CH), jnp.int32),
        pltpu.VMEM((3 * BPW, D), jnp.float32),
        pltpu.SemaphoreType.DMA,
        pltpu.SemaphoreType.DMA,
    ],
)
def _sc_seq_gather(din_hbm, ids1_hbm, ids3_hbm, pos_hbm,
                   tid_hbm, oid_hbm,
                   s1_hbm, s3_hbm, tgtf_hbm, othf_hbm,
                   pos_v, ids1_v, ids3_v, rows_v,
                   tid_v, oid_v, to_v,
                   sem, sem2):
    w = _wid()
    pltpu.sync_copy(pos_hbm.at[w], pos_v)
    pltpu.sync_copy(tid_hbm.at[w], tid_v)
    pltpu.sync_copy(oid_hbm.at[w], oid_v)

    for c in range(2):
        pltpu.async_copy(din_hbm.at[tid_v.at[c]],
                         to_v.at[pl.ds(c * CH, CH)], sem)
    pltpu.async_copy(din_hbm.at[oid_v.at[0]], to_v.at[pl.ds(2 * CH, CH)], sem)

    def fire_ids(c, carry):
        pltpu.async_copy(ids1_hbm.at[pos_v.at[c]], ids1_v.at[c], sem2)
        pltpu.async_copy(ids3_hbm.at[pos_v.at[c]], ids3_v.at[c], sem2)
        return carry
    lax.fori_loop(0, NCH, fire_ids, 0)
    pltpu.make_async_copy(pos_hbm.at[w], ids1_v, sem2).wait()
    pltpu.make_async_copy(pos_hbm.at[w], ids3_v, sem2).wait()

    for i1, out_hbm in ((0, s1_hbm), (1, s3_hbm)):
        ids_v = ids1_v if i1 == 0 else ids3_v
        for h in range(2):
            def fire_rows(c, carry, ids_v=ids_v, h=h):
                pltpu.async_copy(din_hbm.at[ids_v.at[c]],
                                 rows_v.at[pl.ds((c - h * (NCH // 2)) * CH, CH)],
                                 sem2)
                return carry
            lax.fori_loop(h * (NCH // 2), (h + 1) * (NCH // 2), fire_rows, 0)
            pltpu.make_async_copy(s1_hbm.at[pl.ds(0, HALF)], rows_v, sem2).wait()
            pltpu.sync_copy(rows_v, out_hbm.at[pl.ds(w * SPW + h * HALF, HALF)])

    pltpu.make_async_copy(tgtf_hbm.at[pl.ds(0, 3 * CH)], to_v, sem).wait()
    pltpu.sync_copy(to_v.at[pl.ds(0, 2 * BPW)],
                    tgtf_hbm.at[pl.ds(w * 2 * BPW, 2 * BPW)])
    pltpu.sync_copy(to_v.at[pl.ds(2 * BPW, BPW)],
                    othf_hbm.at[pl.ds(w * BPW, BPW)])


@functools.partial(
    pl.kernel,
    out_type=(
        jax.ShapeDtypeStruct((B * 26, D), jnp.float32),
        jax.ShapeDtypeStruct((B * 26, D), jnp.float32),
    ),
    mesh=_mesh,
    compiler_params=_sc_params,
    scratch_types=[
        pltpu.VMEM((WPT, CH), jnp.int32),
        pltpu.VMEM((WPT, CH), jnp.int32),
        pltpu.VMEM((WSPW, D), jnp.float32),
        pltpu.SemaphoreType.DMA,
    ],
)
def _sc_table_gather(wide_t_hbm, deep_t_hbm, wid_hbm, did_hbm,
                     widef_hbm, deepf_hbm,
                     wid_v, did_v, rows_v, sem):
    w = _wid()
    pltpu.sync_copy(wid_hbm.at[w], wid_v)
    pltpu.sync_copy(did_hbm.at[w], did_v)

    for j in range(2):
        ids_v = wid_v if j == 0 else did_v
        table = wide_t_hbm if j == 0 else deep_t_hbm
        out_hbm = widef_hbm if j == 0 else deepf_hbm

        def fire(c, carry, ids_v=ids_v, table=table):
            pltpu.async_copy(table.at[ids_v.at[c]],
                             rows_v.at[pl.ds(c * CH, CH)], sem)
            return carry
        lax.fori_loop(0, WPT, fire, 0)
        pltpu.make_async_copy(out_hbm.at[pl.ds(0, WSPW)], rows_v, sem).wait()
        pltpu.sync_copy(rows_v, out_hbm.at[pl.ds(w * WSPW, WSPW)])


BB = 256
GRID = B // BB
BB2 = 1024
GRID2 = B // BB2


def _dice_k(x, alpha):
    p = jax.nn.sigmoid(x)
    return p * x + (1.0 - p) * alpha * x


def _rep_rows(x, n):
    return jnp.broadcast_to(x[:, None, :], (x.shape[0], n, x.shape[1])
                            ).reshape(x.shape[0] * n, x.shape[1])


def _tc_att_body(s1p_ref, s3p_ref, q1t_ref, q2t_ref, oth_ref, len_ref,
                 k1_ref, k2_ref, k3_ref, k4_ref, kaq1_ref, kaq2_ref,
                 ab1t_ref, aa1t_ref, kw2_ref, ab2t_ref, aa2t_ref, kw3_ref,
                 rmat_ref, smat_ref,
                 mb0_ref, mb1_ref, mb2_ref, mb3_ref, mb4_ref,
                 mpb1_ref, mpa1_ref, mw2_ref, mpb2_ref, mpa2_ref, mw3_ref,
                 scal_ref, out_ref):
    f32 = jnp.float32
    dot = functools.partial(jnp.dot, preferred_element_type=f32)
    s1p = s1p_ref[...]
    s3p = s3p_ref[...]
    q1t = q1t_ref[...]
    q2t = q2t_ref[...]

    q1p = _rep_rows(q1t, PR)
    q2p = _rep_rows(q2t, PR)
    qs1p = q1p * s1p
    qs3p = q2p * s3p

    h = (dot(s1p, k1_ref[...]) + dot(s3p, k2_ref[...])
         + dot(qs1p, k3_ref[...]) + dot(qs3p, k4_ref[...]))
    cqt = dot(q1t, kaq1_ref[...]) + dot(q2t, kaq2_ref[...])
    h = h + _rep_rows(cqt, PR) + ab1t_ref[...]
    h = _dice_k(h, aa1t_ref[...])
    h2 = _dice_k(dot(h, kw2_ref[...]) + ab2t_ref[...], aa2t_ref[...])
    scores = dot(h2, kw3_ref[...]) + scal_ref[0, 0]

    row8 = lax.broadcasted_iota(jnp.int32, (BB * PR, 8), 0) % PR
    colj = lax.broadcasted_iota(jnp.int32, (BB * PR, 8), 1)
    lenp = _rep_rows(len_ref[...], PR)
    mask = (row8 * 8 + colj) < lenp
    scores = jnp.where(mask, scores, -1e9)

    mglob = jnp.max(scores)
    e = jnp.exp(scores - mglob)
    rs = jnp.sum(e, axis=1, keepdims=True)
    denom = jnp.sum(rs.reshape(BB, PR, 1), axis=1)
    wp = e / _rep_rows(denom, PR)

    wE = dot(wp, rmat_ref[...])
    pe1 = jnp.sum((wE * s1p).reshape(BB, PR, 128), axis=1)
    pe3 = jnp.sum((wE * s3p).reshape(BB, PR, 128), axis=1)
    pooled1 = dot(pe1, smat_ref[...])
    pooled3 = dot(pe3, smat_ref[...])

    q1 = q1t[:, :D]
    q2 = q2t[:, :D]
    z = (dot(oth_ref[...], mb0_ref[...]) + dot(pooled1, mb1_ref[...])
         + dot(pooled3, mb2_ref[...]) + dot(q1, mb3_ref[...])
         + dot(q2, mb4_ref[...]) + mpb1_ref[...])
    z = _dice_k(z, mpa1_ref[...])
    z = _dice_k(dot(z, mw2_ref[...]) + mpb2_ref[...], mpa2_ref[...])
    out_ref[...] = jnp.sum(z * mw3_ref[...], axis=-1)[:, None]


def _tc_fin_body(dino_ref, widef_ref, deepf_ref,
                 lrw_ref, dw1_ref, db1_ref, dw2_ref, db2_ref, dw3_ref,
                 scal_ref, out_ref):
    f32 = jnp.float32
    dot = functools.partial(jnp.dot, preferred_element_type=f32)
    lr_o = jnp.sum(widef_ref[...] * lrw_ref[...], axis=-1, keepdims=True)

    hd = jnp.maximum(dot(deepf_ref[...], dw1_ref[...]) + db1_ref[...], 0.0)
    hd = jnp.maximum(dot(hd, dw2_ref[...]) + db2_ref[...], 0.0)
    deep_o = jnp.sum(hd * dw3_ref[...], axis=-1, keepdims=True)

    bias = scal_ref[0, 1] + scal_ref[0, 2] + scal_ref[0, 3]
    out_ref[...] = jax.nn.sigmoid(dino_ref[...] + lr_o + deep_o + bias)


def _full(shape):
    n = len(shape)
    return pl.BlockSpec(shape, lambda i, n=n: (0,) * n)


def kernel(params, seq_ids_1, seq_ids_3, cu_seqlens, target_ids, other_ids,
           wide_ids, deep_ids):
    f32 = jnp.float32
    cu = cu_seqlens.astype(jnp.int32)
    lengths = (cu[1:] - cu[:-1]).reshape(B, 1)
    total = seq_ids_1.shape[0]
    pos = jnp.clip(cu[:-1, None] + jnp.arange(TP, dtype=jnp.int32),
                   0, total - 1)
    pos3d = pos.reshape(NW, NCH, CH)

    s1f, s3f, tgtf, othf = _sc_seq_gather(
        params['din_table'], seq_ids_1, seq_ids_3, pos3d,
        target_ids.reshape(NW, 2 * BPW // CH, CH),
        other_ids.reshape(NW, BPW // CH, CH))

    widef, deepf = _sc_table_gather(
        params['wide_table'], params['deep_table'],
        wide_ids.reshape(NW, WPT, CH), deep_ids.reshape(NW, WPT, CH))

    s1p = s1f.reshape(B * PR, 128)
    s3p = s3f.reshape(B * PR, 128)
    tgt = tgtf.reshape(B, 2, D)
    q1t = jnp.tile(tgt[:, 0, :], (1, 128 // D))
    q2t = jnp.tile(tgt[:, 1, :], (1, 128 // D))
    oth = othf
    widef2 = widef.reshape(B, 26 * D)
    deepf2 = deepf.reshape(B, 26 * D)

    W1 = params['att_W1']
    W1q, W1s, W1d, W1m = W1[0:32], W1[32:64], W1[64:96], W1[96:128]
    As = W1s - W1d
    Aq = W1q + W1d
    eye8 = jnp.eye(128 // D, dtype=f32)
    krn = lambda wgt: jnp.kron(eye8, wgt)
    tl = lambda v: jnp.tile(v.reshape(1, -1), (1, 128 // D))

    mlpW1 = params['mlp_W1']
    mb = [mlpW1[i * D:(i + 1) * D] for i in range(5)]

    r1 = lambda v: v.reshape(1, -1)
    scal = jnp.stack([params['att_b3'][0], params['mlp_b3'][0],
                      params['lr_b'][0], params['deep_b3'][0]]).reshape(1, 4)

    att_ins = [
        krn(As[:D]), krn(As[D:]), krn(W1m[:D]), krn(W1m[D:]),
        krn(Aq[:D]), krn(Aq[D:]),
        tl(params['att_b1']), tl(params['att_a1']), krn(params['att_W2']),
        tl(params['att_b2']), tl(params['att_a2']), krn(params['att_W3']),
        jnp.kron(eye8, jnp.ones((1, D), f32)),
        jnp.tile(jnp.eye(D, dtype=f32), (128 // D, 1)),
        mb[0], mb[1], mb[2], mb[3], mb[4],
        r1(params['mlp_b1']), r1(params['mlp_a1']), params['mlp_W2'],
        r1(params['mlp_b2']), r1(params['mlp_a2']), r1(params['mlp_W3'][:, 0]),
        scal,
    ]

    att_specs = [
        pl.BlockSpec((BB * PR, 128), lambda i: (i, 0)),
        pl.BlockSpec((BB * PR, 128), lambda i: (i, 0)),
        pl.BlockSpec((BB, 128), lambda i: (i, 0)),
        pl.BlockSpec((BB, 128), lambda i: (i, 0)),
        pl.BlockSpec((BB, D), lambda i: (i, 0)),
        pl.BlockSpec((BB, 1), lambda i: (i, 0)),
    ] + [_full(w.shape) for w in att_ins]

    din_o = pl.pallas_call(
        _tc_att_body,
        grid=(GRID,),
        in_specs=att_specs,
        out_specs=pl.BlockSpec((BB, 1), lambda i: (i, 0)),
        out_shape=jax.ShapeDtypeStruct((B, 1), jnp.float32),
    )(s1p, s3p, q1t, q2t, oth, lengths, *att_ins)

    fin_ins = [
        r1(params['lr_w'][:, 0]),
        params['deep_W1'], r1(params['deep_b1']), params['deep_W2'],
        r1(params['deep_b2']), r1(params['deep_W3'][:, 0]),
        scal,
    ]
    fin_specs = [
        pl.BlockSpec((BB2, 1), lambda i: (i, 0)),
        pl.BlockSpec((BB2, 26 * D), lambda i: (i, 0)),
        pl.BlockSpec((BB2, 26 * D), lambda i: (i, 0)),
    ] + [_full(w.shape) for w in fin_ins]

    out = pl.pallas_call(
        _tc_fin_body,
        grid=(GRID2,),
        in_specs=fin_specs,
        out_specs=pl.BlockSpec((BB2, 1), lambda i: (i, 0)),
        out_shape=jax.ShapeDtypeStruct((B, 1), jnp.float32),
    )(din_o, widef2, deepf2, *fin_ins)
    return out
